# baseline (speedup 1.0000x reference)
import functools

import jax
import jax.numpy as jnp
from jax import lax
from jax.experimental import pallas as pl
from jax.experimental.pallas import tpu as pltpu
from jax.experimental.pallas import tpu_sc as plsc

N = 1024
D = 64
M = 1000
ND_USE = 48
EPS = 1e-8
NPAIRS = N * (N - 1) // 2

NC = 2
NS = 16
LANES = 16
NW = NC * NS
ROWS_PER_W = N // NW

_sc_mesh = plsc.VectorSubcoreMesh(core_axis_name="c", subcore_axis_name="s")


SCR_ROWS = N * N // 128
CHUNK = ROWS_PER_W * N // 128


@functools.partial(
    pl.kernel,
    mesh=_sc_mesh,
    compiler_params=pltpu.CompilerParams(
        use_tc_tiling_on_sc=False, needs_layout_passes=False),
    out_type=jax.ShapeDtypeStruct((SCR_ROWS, 128), jnp.float32),
    scratch_types=[
        pltpu.VMEM((N,), jnp.int32),
        pltpu.VMEM((ROWS_PER_W,), jnp.int32),
        pltpu.VMEM((ROWS_PER_W, M), jnp.float32),
        pltpu.VMEM((CHUNK, 128), jnp.float32),
        pltpu.SemaphoreType.DMA,
        pltpu.SemaphoreType.DMA,
        pltpu.SemaphoreType.DMA,
        pltpu.SemaphoreType.DMA,
    ],
)
def _sc_affinity(yt_hbm, lookup_hbm, out_hbm, yt_v, idx_v, rows_v, a_v,
                 sem_yt, sem_idx, sem_rows, sem_wb):
    wid = lax.axis_index("s") * NC + lax.axis_index("c")
    base_a = wid * 16
    base_b = 1008 - wid * 16
    cp_yt = pltpu.async_copy(yt_hbm, yt_v, sem_yt)
    cp_i1 = pltpu.async_copy(yt_hbm.at[pl.ds(base_a, 16)],
                             idx_v.at[pl.ds(0, 16)], sem_idx)
    cp_i2 = pltpu.async_copy(yt_hbm.at[pl.ds(base_b, 16)],
                             idx_v.at[pl.ds(16, 16)], sem_idx)
    cp_i1.wait()
    cp_i2.wait()
    cp_rows = pltpu.async_copy(lookup_hbm.at[idx_v], rows_v, sem_rows)
    cp_yt.wait()
    cp_rows.wait()

    def col_body_a(jb, carry):
        cidx = yt_v[pl.ds(jb * LANES, LANES)]
        u_base = 8 * (jb // 8)
        c_loc = LANES * (jb % 8)

        @plsc.parallel_loop(0, 16, unroll=8)
        def row_body_a(r):
            row_splat = jnp.full((LANES,), r, jnp.int32)
            vals = plsc.load_gather(rows_v, [row_splat, cidx])
            u_loc = u_base + 64 * (r // 8) + (r % 8)
            a_v[u_loc, pl.ds(c_loc, LANES)] = vals

        return carry

    def col_body_b(jb, carry):
        cidx = yt_v[pl.ds(jb * LANES, LANES)]
        u_base = 8 * (jb // 8)
        c_loc = LANES * (jb % 8)

        @plsc.parallel_loop(0, 16, unroll=8)
        def row_body_b(r):
            row_splat = jnp.full((LANES,), 16 + r, jnp.int32)
            vals = plsc.load_gather(rows_v, [row_splat, cidx])
            u_loc = 128 + u_base + 64 * (r // 8) + (r % 8)
            a_v[u_loc, pl.ds(c_loc, LANES)] = vals

        return carry

    lax.fori_loop(wid, N // LANES, col_body_a, 0)
    wb_a = pltpu.async_copy(a_v.at[pl.ds(0, 128)],
                            out_hbm.at[pl.ds(wid * 128, 128)], sem_wb)
    lax.fori_loop(63 - wid, N // LANES, col_body_b, 0)
    wb_b = pltpu.async_copy(a_v.at[pl.ds(128, 128)],
                            out_hbm.at[pl.ds(64 * (126 - 2 * wid), 128)], sem_wb)
    wb_a.wait()
    wb_b.wait()


NCHUNK = 2
CROWS = SCR_ROWS // NCHUNK
GROWS = N // NCHUNK


def _tc_loss_body(yp_ref, a_ref, out_ref, acc):
    cc = pl.program_id(0)
    yp = yp_ref[:]
    col = lax.broadcasted_iota(jnp.int32, (N, D), 1)
    ypm = jnp.where(col < ND_USE, yp, 0.0)
    nrm = jnp.sqrt(jnp.sum(ypm * ypm, axis=1, keepdims=True))
    yn = ypm * (1.0 / jnp.maximum(nrm, EPS))
    ypb = yp_ref[pl.ds(cc * GROWS, GROWS), :]
    colb = lax.broadcasted_iota(jnp.int32, (GROWS, D), 1)
    ypbm = jnp.where(colb < ND_USE, ypb, 0.0)
    nrmb = jnp.sqrt(jnp.sum(ypbm * ypbm, axis=1, keepdims=True))
    yn_blk = ypbm * (1.0 / jnp.maximum(nrmb, EPS))
    blocks = []
    for tj in range(8):
        b = yn[128 * tj:128 * (tj + 1), :]
        m = lax.dot_general(yn_blk, b, (((1,), (1,)), ((), ())),
                            preferred_element_type=jnp.float32)
        blocks.append(m.reshape(GROWS // 8, 1, 8, 128))
    gscr = jnp.concatenate(blocks, axis=1).reshape(CROWS, 128)
    u = lax.broadcasted_iota(jnp.int32, (CROWS, 128), 0) + cc * CROWS
    c = lax.broadcasted_iota(jnp.int32, (CROWS, 128), 1)
    ii = 8 * (u // 64) + (u % 8)
    jj = 128 * ((u // 8) % 8) + c
    diff = jnp.abs(gscr - a_ref[:])
    part = jnp.sum(jnp.where(jj > ii, diff, 0.0))

    @pl.when(cc == 0)
    def _():
        acc[0] = part

    @pl.when(cc == NCHUNK - 1)
    def _():
        out_ref[0, 0] = (acc[0] + part) * (1.0 / NPAIRS)


_tc_loss = pl.pallas_call(
    _tc_loss_body,
    grid=(NCHUNK,),
    out_shape=jax.ShapeDtypeStruct((1, 1), jnp.float32),
    in_specs=[
        pl.BlockSpec((N, D), lambda cc: (0, 0)),
        pl.BlockSpec((CROWS, 128), lambda cc: (cc, 0)),
    ],
    out_specs=pl.BlockSpec(memory_space=pltpu.SMEM),
    scratch_shapes=[
        pltpu.SMEM((1,), jnp.float32),
    ],
)


@jax.jit
def kernel(y_true, y_pred, lookup):
    yt = y_true.astype(jnp.int32)
    a = _sc_affinity(yt, lookup)
    loss = _tc_loss(y_pred, a)
    return loss[0, 0]


# speedup vs baseline: 1.1352x; 1.1352x over previous
import functools

import jax
import jax.numpy as jnp
from jax import lax
from jax.experimental import pallas as pl
from jax.experimental.pallas import tpu as pltpu
from jax.experimental.pallas import tpu_sc as plsc

N = 1024
D = 64
M = 1000
ND_USE = 48
EPS = 1e-8
NPAIRS = N * (N - 1) // 2

NC = 2
NS = 16
LANES = 16
NW = NC * NS
ROWS_PER_W = N // NW

_sc_mesh = plsc.VectorSubcoreMesh(core_axis_name="c", subcore_axis_name="s")


SCR_ROWS = N * N // 128
CHUNK = ROWS_PER_W * N // 128


@functools.partial(
    pl.kernel,
    mesh=_sc_mesh,
    compiler_params=pltpu.CompilerParams(
        use_tc_tiling_on_sc=False, needs_layout_passes=False),
    out_type=jax.ShapeDtypeStruct((SCR_ROWS, 128), jnp.float32),
    scratch_types=[
        pltpu.VMEM((N,), jnp.int32),
        pltpu.VMEM((ROWS_PER_W,), jnp.int32),
        pltpu.VMEM((ROWS_PER_W, M), jnp.float32),
        pltpu.VMEM((CHUNK, 128), jnp.float32),
        pltpu.SemaphoreType.DMA,
        pltpu.SemaphoreType.DMA,
        pltpu.SemaphoreType.DMA,
        pltpu.SemaphoreType.DMA,
    ],
)
def _sc_affinity(yt_hbm, lookup_hbm, out_hbm, yt_v, idx_v, rows_v, a_v,
                 sem_yt, sem_idx, sem_rows, sem_wb):
    wid = lax.axis_index("s") * NC + lax.axis_index("c")
    base_a = wid * 16
    base_b = 1008 - wid * 16
    cp_yt = pltpu.async_copy(yt_hbm, yt_v, sem_yt)
    cp_i1 = pltpu.async_copy(yt_hbm.at[pl.ds(base_a, 16)],
                             idx_v.at[pl.ds(0, 16)], sem_idx)
    cp_i2 = pltpu.async_copy(yt_hbm.at[pl.ds(base_b, 16)],
                             idx_v.at[pl.ds(16, 16)], sem_idx)
    cp_i1.wait()
    cp_i2.wait()
    cp_rows = pltpu.async_copy(lookup_hbm.at[idx_v], rows_v, sem_rows)
    cp_yt.wait()
    cp_rows.wait()

    def col_body_a(jb, carry):
        cidx = yt_v[pl.ds(jb * LANES, LANES)]
        u_base = 8 * (jb // 8)
        c_loc = LANES * (jb % 8)

        @plsc.parallel_loop(0, 16, unroll=8)
        def row_body_a(r):
            row_splat = jnp.full((LANES,), r, jnp.int32)
            vals = plsc.load_gather(rows_v, [row_splat, cidx])
            u_loc = u_base + 64 * (r // 8) + (r % 8)
            a_v[u_loc, pl.ds(c_loc, LANES)] = vals

        return carry

    def col_body_b(jb, carry):
        cidx = yt_v[pl.ds(jb * LANES, LANES)]
        u_base = 8 * (jb // 8)
        c_loc = LANES * (jb % 8)

        @plsc.parallel_loop(0, 16, unroll=8)
        def row_body_b(r):
            row_splat = jnp.full((LANES,), 16 + r, jnp.int32)
            vals = plsc.load_gather(rows_v, [row_splat, cidx])
            u_loc = 128 + u_base + 64 * (r // 8) + (r % 8)
            a_v[u_loc, pl.ds(c_loc, LANES)] = vals

        return carry

    lax.fori_loop(wid, N // LANES, col_body_a, 0)
    wb_a = pltpu.async_copy(a_v.at[pl.ds(0, 128)],
                            out_hbm.at[pl.ds(wid * 128, 128)], sem_wb)
    lax.fori_loop(63 - wid, N // LANES, col_body_b, 0)
    wb_b = pltpu.async_copy(a_v.at[pl.ds(128, 128)],
                            out_hbm.at[pl.ds(64 * (126 - 2 * wid), 128)], sem_wb)
    wb_a.wait()
    wb_b.wait()


def _tc_loss_body(yp_ref, a_ref, out_ref):
    yp = yp_ref[:]
    col = lax.broadcasted_iota(jnp.int32, (N, D), 1)
    ypm = jnp.where(col < ND_USE, yp, 0.0)
    nrm = jnp.sqrt(jnp.sum(ypm * ypm, axis=1, keepdims=True))
    yn = ypm * (1.0 / jnp.maximum(nrm, EPS))
    blocks = []
    for tj in range(8):
        b = yn[128 * tj:128 * (tj + 1), :]
        m = lax.dot_general(yn, b, (((1,), (1,)), ((), ())),
                            preferred_element_type=jnp.float32)
        blocks.append(m.reshape(N // 8, 1, 8, 128))
    gscr = jnp.concatenate(blocks, axis=1).reshape(SCR_ROWS, 128)
    u = lax.broadcasted_iota(jnp.int32, (SCR_ROWS, 128), 0)
    c = lax.broadcasted_iota(jnp.int32, (SCR_ROWS, 128), 1)
    ii = 8 * (u // 64) + (u % 8)
    jj = 128 * ((u // 8) % 8) + c
    diff = jnp.abs(gscr - a_ref[:])
    s = jnp.sum(jnp.where(jj > ii, diff, 0.0))
    out_ref[0, 0] = s * (1.0 / NPAIRS)


_tc_loss = pl.pallas_call(
    _tc_loss_body,
    out_shape=jax.ShapeDtypeStruct((1, 1), jnp.float32),
    in_specs=[
        pl.BlockSpec(memory_space=pltpu.VMEM),
        pl.BlockSpec(memory_space=pltpu.VMEM),
    ],
    out_specs=pl.BlockSpec(memory_space=pltpu.SMEM),
)


@jax.jit
def kernel(y_true, y_pred, lookup):
    yt = y_true.astype(jnp.int32)
    a = _sc_affinity(yt, lookup)
    loss = _tc_loss(y_pred, a)
    return loss[0, 0]
